# Initial kernel scaffold; baseline (speedup 1.0000x reference)
import jax
import jax.numpy as jnp
from jax.experimental import pallas as pl


def kernel(tokens, t_idx, time_table, type_image, ln_gamma, ln_beta):
    raise NotImplementedError("write your pallas kernel here")


# speedup vs baseline: 5.3178x; 5.3178x over previous
import jax
import jax.numpy as jnp
from jax.experimental import pallas as pl
from jax.experimental.pallas import tpu as pltpu

_B, _T, _D, _H = 1024, 200, 512, 4
_TIME_OFFSET = _H - 1
_LN_EPS = 1e-5
_HP = 128
_BB = 8


def _ln_body(oh_ref, tab_ref, type_ref, g_ref, b_ref, x_ref, o_ref):
    pe = jnp.dot(oh_ref[...], tab_ref[...], preferred_element_type=jnp.float32)
    bias = pe + type_ref[...]
    x = x_ref[...] + bias[None, :, :]
    mean = jnp.mean(x, axis=-1, keepdims=True)
    xc = x - mean
    var = jnp.mean(xc * xc, axis=-1, keepdims=True)
    o_ref[...] = xc * jax.lax.rsqrt(var + _LN_EPS) * g_ref[...] + b_ref[...]


@jax.jit
def kernel(tokens, t_idx, time_table, type_image, ln_gamma, ln_beta):
    idx = jnp.clip(t_idx.astype(jnp.int32) + _TIME_OFFSET, 0, _H - 1)
    onehot = (idx[:, None] == jnp.arange(_HP, dtype=jnp.int32)[None, :])
    onehot = onehot.astype(jnp.float32)
    tab = jnp.zeros((_HP, _D), jnp.float32).at[:_H].set(time_table)
    type_row = type_image.reshape(1, _D)
    g = ln_gamma.reshape(1, _D)
    b = ln_beta.reshape(1, _D)

    grid = (_B // _BB,)
    return pl.pallas_call(
        _ln_body,
        grid=grid,
        in_specs=[
            pl.BlockSpec((_T, _HP), lambda i: (0, 0)),
            pl.BlockSpec((_HP, _D), lambda i: (0, 0)),
            pl.BlockSpec((1, _D), lambda i: (0, 0)),
            pl.BlockSpec((1, _D), lambda i: (0, 0)),
            pl.BlockSpec((1, _D), lambda i: (0, 0)),
            pl.BlockSpec((_BB, _T, _D), lambda i: (i, 0, 0)),
        ],
        out_specs=pl.BlockSpec((_BB, _T, _D), lambda i: (i, 0, 0)),
        out_shape=jax.ShapeDtypeStruct((_B, _T, _D), jnp.float32),
        compiler_params=pltpu.CompilerParams(
            dimension_semantics=("parallel",),
        ),
    )(onehot, tab, type_row, g, b, tokens)


# speedup vs baseline: 5.8313x; 1.0966x over previous
import jax
import jax.numpy as jnp
from jax.experimental import pallas as pl
from jax.experimental.pallas import tpu as pltpu

_B, _T, _D, _H = 1024, 200, 512, 4
_TIME_OFFSET = _H - 1
_LN_EPS = 1e-5
_HP = 128
_BB = 16


def _ln_body(oh_ref, tab_ref, type_ref, g_ref, b_ref, x_ref, o_ref):
    pe = jnp.dot(oh_ref[...], tab_ref[...], preferred_element_type=jnp.float32)
    bias = pe + type_ref[...]
    x = x_ref[...] + bias[None, :, :]
    mean = jnp.mean(x, axis=-1, keepdims=True)
    xc = x - mean
    var = jnp.mean(xc * xc, axis=-1, keepdims=True)
    o_ref[...] = xc * jax.lax.rsqrt(var + _LN_EPS) * g_ref[...] + b_ref[...]


@jax.jit
def kernel(tokens, t_idx, time_table, type_image, ln_gamma, ln_beta):
    idx = jnp.clip(t_idx.astype(jnp.int32) + _TIME_OFFSET, 0, _H - 1)
    onehot = (idx[:, None] == jnp.arange(_HP, dtype=jnp.int32)[None, :])
    onehot = onehot.astype(jnp.float32)
    tab = jnp.zeros((_HP, _D), jnp.float32).at[:_H].set(time_table)
    type_row = type_image.reshape(1, _D)
    g = ln_gamma.reshape(1, _D)
    b = ln_beta.reshape(1, _D)

    grid = (_B // _BB,)
    return pl.pallas_call(
        _ln_body,
        grid=grid,
        in_specs=[
            pl.BlockSpec((_T, _HP), lambda i: (0, 0)),
            pl.BlockSpec((_HP, _D), lambda i: (0, 0)),
            pl.BlockSpec((1, _D), lambda i: (0, 0)),
            pl.BlockSpec((1, _D), lambda i: (0, 0)),
            pl.BlockSpec((1, _D), lambda i: (0, 0)),
            pl.BlockSpec((_BB, _T, _D), lambda i: (i, 0, 0)),
        ],
        out_specs=pl.BlockSpec((_BB, _T, _D), lambda i: (i, 0, 0)),
        out_shape=jax.ShapeDtypeStruct((_B, _T, _D), jnp.float32),
        compiler_params=pltpu.CompilerParams(
            dimension_semantics=("parallel",),
        ),
    )(onehot, tab, type_row, g, b, tokens)
